# baseline (speedup 1.0000x reference)
import functools

import jax
import jax.numpy as jnp
from jax import lax
from jax.experimental import pallas as pl
from jax.experimental.pallas import tpu as pltpu
from jax.experimental.pallas import tpu_sc as plsc

N = 16384
HID = 1024
D = 256
L = 16
FLAT = N * (HID // D)
NC, NS = 2, 16
NW = NC * NS
FR_W = FLAT // NW
C = 32
NCH = FR_W // C
NBUF = 4


def _sc_embed_add(x_f, idx_all, tab_all):
    mesh = plsc.VectorSubcoreMesh(core_axis_name="c", subcore_axis_name="s")

    @functools.partial(
        pl.kernel,
        mesh=mesh,
        out_type=jax.ShapeDtypeStruct((FLAT, D), jnp.float32),
        scratch_types=(
            [pltpu.VMEM((FR_W,), jnp.int32)]
            + [pltpu.VMEM((C, D), jnp.float32) for _ in range(NBUF)]
            + [pltpu.VMEM((C, D), jnp.float32) for _ in range(NBUF)]
            + [pltpu.SemaphoreType.DMA for _ in range(NBUF)]
            + [pltpu.SemaphoreType.DMA for _ in range(NBUF)]
            + [pltpu.SemaphoreType.DMA for _ in range(NBUF)]
        ),
    )
    def k(x_hbm, idx_hbm, tab_hbm, out_hbm, idx_v, *bufs_sems):
        xb = bufs_sems[0:NBUF]
        gb = bufs_sems[NBUF:2 * NBUF]
        sx = bufs_sems[2 * NBUF:3 * NBUF]
        sg = bufs_sems[3 * NBUF:4 * NBUF]
        so = bufs_sems[4 * NBUF:5 * NBUF]
        wid = lax.axis_index("s") * NC + lax.axis_index("c")
        base = wid * FR_W
        pltpu.sync_copy(idx_hbm.at[pl.ds(base, FR_W)], idx_v)

        def issue_in(ci, b):
            pltpu.async_copy(x_hbm.at[pl.ds(base + ci * C, C)], xb[b], sx[b])
            pltpu.async_copy(
                tab_hbm.at[idx_v.at[pl.ds(ci * C, C)]], gb[b], sg[b])

        def wait_in(b):
            pltpu.make_async_copy(x_hbm.at[pl.ds(0, C)], xb[b], sx[b]).wait()
            pltpu.make_async_copy(
                tab_hbm.at[idx_v.at[pl.ds(0, C)]], gb[b], sg[b]).wait()

        def accumulate(b):
            def row_body(i, _):
                for j in range(D // L):
                    g = gb[b][i, pl.ds(j * L, L)]
                    plsc.addupdate(xb[b].at[i, pl.ds(j * L, L)], g)
                return 0

            lax.fori_loop(0, C, row_body, 0)

        def issue_out(ci, b):
            pltpu.async_copy(xb[b], out_hbm.at[pl.ds(base + ci * C, C)], so[b])

        def wait_out(b):
            pltpu.make_async_copy(xb[b], out_hbm.at[pl.ds(0, C)], so[b]).wait()

        issue_in(0, 0)
        for b in range(NBUF):
            if b + 1 < NBUF:
                issue_in(b + 1, b + 1)
            else:
                wait_out(0)
                issue_in(NBUF, 0)
            wait_in(b)
            accumulate(b)
            issue_out(b, b)

        def body(g4, _):
            g = g4 * NBUF
            for b in range(NBUF):
                ci = g + b
                bn = (b + 1) % NBUF
                cin = jnp.minimum(ci + 1, NCH - 1)
                wait_out(bn)
                issue_in(cin, bn)
                wait_in(b)
                accumulate(b)
                issue_out(ci, b)
            return 0

        lax.fori_loop(1, NCH // NBUF, body, 0)

        wait_in(0)
        for b in range(1, NBUF):
            wait_out(b)

    return k(x_f, idx_all, tab_all)


def kernel(x, chan_ind, spat_ind, embed_channel, embed_spatial):
    c0 = chan_ind[:, 0].astype(jnp.int32)
    c1 = chan_ind[:, 1].astype(jnp.int32)
    s0 = spat_ind[:, 0].astype(jnp.int32)
    s1 = spat_ind[:, 1].astype(jnp.int32)
    n_ch = embed_channel.shape[0]
    idx_all = jnp.stack([c0, c1, s0 + n_ch, s1 + n_ch], axis=1).reshape(-1)
    tab_all = jnp.concatenate(
        [embed_channel.astype(jnp.float32), embed_spatial.astype(jnp.float32)],
        axis=0)
    x_f = x.reshape(FLAT, D)
    out_f = _sc_embed_add(x_f, idx_all, tab_all)
    return out_f.reshape(N, HID)


# speedup vs baseline: 1.6925x; 1.6925x over previous
import functools

import jax
import jax.numpy as jnp
from jax import lax
from jax.experimental import pallas as pl
from jax.experimental.pallas import tpu as pltpu
from jax.experimental.pallas import tpu_sc as plsc

N = 16384
HID = 1024
D = 256
P = HID // D
L = 16
NC, NS = 2, 16
NW = NC * NS
ROWS_W = N // NW
CF = 8
C = CF * P
NCH = ROWS_W // CF
NBUF = 4


def _sc_embed_add(x, idx_all, tab_all):
    mesh = plsc.VectorSubcoreMesh(core_axis_name="c", subcore_axis_name="s")

    @functools.partial(
        pl.kernel,
        mesh=mesh,
        out_type=jax.ShapeDtypeStruct((N, HID), jnp.float32),
        scratch_types=(
            [pltpu.VMEM((ROWS_W * P,), jnp.int32)]
            + [pltpu.VMEM((CF, HID), jnp.float32) for _ in range(NBUF)]
            + [pltpu.VMEM((C, D), jnp.float32) for _ in range(NBUF)]
            + [pltpu.SemaphoreType.DMA for _ in range(NBUF)]
            + [pltpu.SemaphoreType.DMA for _ in range(NBUF)]
            + [pltpu.SemaphoreType.DMA for _ in range(NBUF)]
        ),
    )
    def k(x_hbm, idx_hbm, tab_hbm, out_hbm, idx_v, *bufs_sems):
        xb = bufs_sems[0:NBUF]
        gb = bufs_sems[NBUF:2 * NBUF]
        sx = bufs_sems[2 * NBUF:3 * NBUF]
        sg = bufs_sems[3 * NBUF:4 * NBUF]
        so = bufs_sems[4 * NBUF:5 * NBUF]
        wid = lax.axis_index("s") * NC + lax.axis_index("c")
        rbase = wid * ROWS_W
        fbase = rbase * P
        pltpu.sync_copy(idx_hbm.at[pl.ds(fbase, ROWS_W * P)], idx_v)

        def issue_in(ci, b):
            pltpu.async_copy(
                x_hbm.at[pl.ds(rbase + ci * CF, CF)], xb[b], sx[b])
            pltpu.async_copy(
                tab_hbm.at[idx_v.at[pl.ds(ci * C, C)]], gb[b], sg[b])

        def wait_in(b):
            pltpu.make_async_copy(x_hbm.at[pl.ds(0, CF)], xb[b], sx[b]).wait()
            pltpu.make_async_copy(
                tab_hbm.at[idx_v.at[pl.ds(0, C)]], gb[b], sg[b]).wait()

        def accumulate(b):
            def row_body(fr, _):
                g0 = fr * P
                for q in range(P):
                    for j in range(D // L):
                        g = gb[b][g0 + q, pl.ds(j * L, L)]
                        plsc.addupdate(
                            xb[b].at[fr, pl.ds(q * D + j * L, L)], g)
                return 0

            lax.fori_loop(0, CF, row_body, 0)

        def issue_out(ci, b):
            pltpu.async_copy(
                xb[b], out_hbm.at[pl.ds(rbase + ci * CF, CF)], so[b])

        def wait_out(b):
            pltpu.make_async_copy(
                xb[b], out_hbm.at[pl.ds(0, CF)], so[b]).wait()

        issue_in(0, 0)
        for b in range(NBUF):
            if b + 1 < NBUF:
                issue_in(b + 1, b + 1)
            else:
                wait_out(0)
                issue_in(NBUF, 0)
            wait_in(b)
            accumulate(b)
            issue_out(b, b)

        def body(g4, _):
            g = g4 * NBUF
            for b in range(NBUF):
                ci = g + b
                bn = (b + 1) % NBUF
                cin = jnp.minimum(ci + 1, NCH - 1)
                wait_out(bn)
                issue_in(cin, bn)
                wait_in(b)
                accumulate(b)
                issue_out(ci, b)
            return 0

        lax.fori_loop(1, NCH // NBUF, body, 0)

        wait_in(0)
        for b in range(1, NBUF):
            wait_out(b)

    return k(x, idx_all, tab_all)


def kernel(x, chan_ind, spat_ind, embed_channel, embed_spatial):
    c0 = chan_ind[:, 0].astype(jnp.int32)
    c1 = chan_ind[:, 1].astype(jnp.int32)
    s0 = spat_ind[:, 0].astype(jnp.int32)
    s1 = spat_ind[:, 1].astype(jnp.int32)
    n_ch = embed_channel.shape[0]
    idx_all = jnp.stack([c0, c1, s0 + n_ch, s1 + n_ch], axis=1).reshape(-1)
    tab_all = jnp.concatenate(
        [embed_channel.astype(jnp.float32), embed_spatial.astype(jnp.float32)],
        axis=0)
    return _sc_embed_add(x, idx_all, tab_all)


# speedup vs baseline: 1.7178x; 1.0150x over previous
import functools

import jax
import jax.numpy as jnp
from jax import lax
from jax.experimental import pallas as pl
from jax.experimental.pallas import tpu as pltpu
from jax.experimental.pallas import tpu_sc as plsc

N = 16384
HID = 1024
D = 256
P = HID // D
L = 16
NC, NS = 2, 16
NW = NC * NS
ROWS_W = N // NW
CF = 8
C = CF * P
NCH = ROWS_W // CF
NBUF = 4


def _sc_embed_add(x, idx_all, tab_all):
    mesh = plsc.VectorSubcoreMesh(core_axis_name="c", subcore_axis_name="s")

    @functools.partial(
        pl.kernel,
        mesh=mesh,
        out_type=jax.ShapeDtypeStruct((N, HID), jnp.float32),
        scratch_types=(
            [pltpu.VMEM((ROWS_W * P,), jnp.int32)]
            + [pltpu.VMEM((CF, HID), jnp.float32) for _ in range(NBUF)]
            + [pltpu.VMEM((C, D), jnp.float32) for _ in range(NBUF)]
            + [pltpu.SemaphoreType.DMA for _ in range(NBUF)]
            + [pltpu.SemaphoreType.DMA for _ in range(NBUF)]
            + [pltpu.SemaphoreType.DMA for _ in range(NBUF)]
        ),
    )
    def k(x_hbm, idx_hbm, tab_hbm, out_hbm, idx_v, *bufs_sems):
        xb = bufs_sems[0:NBUF]
        gb = bufs_sems[NBUF:2 * NBUF]
        sx = bufs_sems[2 * NBUF:3 * NBUF]
        sg = bufs_sems[3 * NBUF:4 * NBUF]
        so = bufs_sems[4 * NBUF:5 * NBUF]
        wid = lax.axis_index("s") * NC + lax.axis_index("c")
        rbase = wid * ROWS_W
        fbase = rbase * P
        pltpu.sync_copy(idx_hbm.at[pl.ds(fbase, ROWS_W * P)], idx_v)

        def issue_in(ci, b):
            pltpu.async_copy(
                x_hbm.at[pl.ds(rbase + ci * CF, CF)], xb[b], sx[b])
            pltpu.async_copy(
                tab_hbm.at[idx_v.at[pl.ds(ci * C, C)]], gb[b], sg[b])

        def wait_in(b):
            pltpu.make_async_copy(x_hbm.at[pl.ds(0, CF)], xb[b], sx[b]).wait()
            pltpu.make_async_copy(
                tab_hbm.at[idx_v.at[pl.ds(0, C)]], gb[b], sg[b]).wait()

        def accumulate(b):
            def row_body(fr, _):
                g0 = fr * P
                for q in range(P):
                    for j in range(D // L):
                        g = gb[b][g0 + q, pl.ds(j * L, L)]
                        plsc.addupdate(
                            xb[b].at[fr, pl.ds(q * D + j * L, L)], g)
                return 0

            lax.fori_loop(0, CF, row_body, 0)

        def issue_out(ci, b):
            pltpu.async_copy(
                xb[b], out_hbm.at[pl.ds(rbase + ci * CF, CF)], so[b])

        def wait_out(b):
            pltpu.make_async_copy(
                xb[b], out_hbm.at[pl.ds(0, CF)], so[b]).wait()

        issue_in(0, 0)
        issue_in(1, 1)
        for b in range(NBUF):
            if b + 2 < NBUF:
                issue_in(b + 2, b + 2)
            else:
                wait_out(b - 2)
                issue_in(b + 2, b - 2)
            wait_in(b)
            accumulate(b)
            issue_out(b, b)

        def body(g4, _):
            g = g4 * NBUF
            for b in range(NBUF):
                ci = g + b
                bn = (b + 2) % NBUF
                cin = jnp.minimum(ci + 2, NCH - 1)
                wait_out(bn)
                issue_in(cin, bn)
                wait_in(b)
                accumulate(b)
                issue_out(ci, b)
            return 0

        lax.fori_loop(1, NCH // NBUF, body, 0)

        wait_in(0)
        wait_in(1)
        for b in range(2, NBUF):
            wait_out(b)

    return k(x, idx_all, tab_all)


def kernel(x, chan_ind, spat_ind, embed_channel, embed_spatial):
    c0 = chan_ind[:, 0].astype(jnp.int32)
    c1 = chan_ind[:, 1].astype(jnp.int32)
    s0 = spat_ind[:, 0].astype(jnp.int32)
    s1 = spat_ind[:, 1].astype(jnp.int32)
    n_ch = embed_channel.shape[0]
    idx_all = jnp.stack([c0, c1, s0 + n_ch, s1 + n_ch], axis=1).reshape(-1)
    tab_all = jnp.concatenate(
        [embed_channel.astype(jnp.float32), embed_spatial.astype(jnp.float32)],
        axis=0)
    return _sc_embed_add(x, idx_all, tab_all)
